# baseline (speedup 1.0000x reference)
import jax
import jax.numpy as jnp
from jax.experimental import pallas as pl
from jax.experimental.pallas import tpu as pltpu

_RADIUS2 = 0.25
_B = 2
_N = 4096
_TM = 512
_T = _N // _TM


_C = 4


def _row_copy(rows_ref, out_ref, sem, b, i, c):
    w = _N // _C
    return pltpu.make_async_copy(
        rows_ref.at[b, i, :, pl.ds(c * w, w)],
        out_ref.at[b, pl.ds(i * _TM, _TM), pl.ds(c * w, w)],
        sem.at[b, i, c],
    )


def _adj_kernel(p_ref, pt_ref, out_ref, rows_ref, sem):
    for b in range(_B):
        pt = pt_ref[b]
        sj_full = jnp.sum(pt * pt, axis=0, keepdims=True)
        for I in range(_T):
            pi = p_ref[b, I * _TM:(I + 1) * _TM, :]
            si = jnp.sum(pi * pi, axis=1, keepdims=True)
            npi = -2.0 * pi
            for J in range(I, _T):
                lo = J * _TM
                m2 = jax.lax.dot_general(
                    npi, pt[:, lo:lo + _TM], (((1,), (0,)), ((), ())),
                    preferred_element_type=jnp.float32)
                dist = (m2 + si) + sj_full[:, lo:lo + _TM]
                v = (dist <= _RADIUS2).astype(jnp.int8)
                rows_ref[b, I, :, lo:lo + _TM] = v
                if J != I:
                    rows_ref[b, J, :, I * _TM:(I + 1) * _TM] = v.T
            for c in range(_C):
                _row_copy(rows_ref, out_ref, sem, b, I, c).start()
    for b in range(_B):
        for I in range(_T):
            for c in range(_C):
                _row_copy(rows_ref, out_ref, sem, b, I, c).wait()


def kernel(batch_points, batch_feats, batch_len):
    pts = batch_points.reshape(_B, _N, 3)
    fts = batch_feats.reshape(_B, _N, batch_feats.shape[-1])
    pts_t = jnp.swapaxes(pts, 1, 2)

    adj8 = pl.pallas_call(
        _adj_kernel,
        in_specs=[
            pl.BlockSpec(memory_space=pltpu.MemorySpace.VMEM),
            pl.BlockSpec(memory_space=pltpu.MemorySpace.VMEM),
        ],
        out_specs=pl.BlockSpec(memory_space=pltpu.MemorySpace.HBM),
        out_shape=jax.ShapeDtypeStruct((_B, _N, _N), jnp.int8),
        scratch_shapes=[
            pltpu.VMEM((_B, _T, _TM, _N), jnp.int8),
            pltpu.SemaphoreType.DMA((_B, _T, _C)),
        ],
    )(pts, pts_t)
    adj = adj8.view(jnp.bool_)
    return adj, pts, fts


# speedup vs baseline: 1.0093x; 1.0093x over previous
import jax
import jax.numpy as jnp
from jax.experimental import pallas as pl
from jax.experimental.pallas import tpu as pltpu

_RADIUS2 = 0.25
_B = 2
_N = 4096
_TM = 256
_T = _N // _TM


def _row_copy(rows_ref, out_ref, sem, b, i):
    return pltpu.make_async_copy(
        rows_ref.at[b, i],
        out_ref.at[b, pl.ds(i * _TM, _TM), :],
        sem.at[b, i],
    )


def _adj_kernel(p_ref, pt_ref, out_ref, rows_ref, sem):
    for b in range(_B):
        pt = pt_ref[b]
        sj_full = jnp.sum(pt * pt, axis=0, keepdims=True)
        for I in range(_T):
            pi = p_ref[b, I * _TM:(I + 1) * _TM, :]
            si = jnp.sum(pi * pi, axis=1, keepdims=True)
            npi = -2.0 * pi
            for J in range(I, _T):
                lo = J * _TM
                m2 = jax.lax.dot_general(
                    npi, pt[:, lo:lo + _TM], (((1,), (0,)), ((), ())),
                    preferred_element_type=jnp.float32)
                dist = (m2 + si) + sj_full[:, lo:lo + _TM]
                v = (dist <= _RADIUS2).astype(jnp.int8)
                rows_ref[b, I, :, lo:lo + _TM] = v
                if J != I:
                    rows_ref[b, J, :, I * _TM:(I + 1) * _TM] = v.T
            _row_copy(rows_ref, out_ref, sem, b, I).start()
    for b in range(_B):
        for I in range(_T):
            _row_copy(rows_ref, out_ref, sem, b, I).wait()


def kernel(batch_points, batch_feats, batch_len):
    pts = batch_points.reshape(_B, _N, 3)
    fts = batch_feats.reshape(_B, _N, batch_feats.shape[-1])
    pts_t = jnp.swapaxes(pts, 1, 2)

    adj8 = pl.pallas_call(
        _adj_kernel,
        in_specs=[
            pl.BlockSpec(memory_space=pltpu.MemorySpace.VMEM),
            pl.BlockSpec(memory_space=pltpu.MemorySpace.VMEM),
        ],
        out_specs=pl.BlockSpec(memory_space=pltpu.MemorySpace.HBM),
        out_shape=jax.ShapeDtypeStruct((_B, _N, _N), jnp.int8),
        scratch_shapes=[
            pltpu.VMEM((_B, _T, _TM, _N), jnp.int8),
            pltpu.SemaphoreType.DMA((_B, _T)),
        ],
    )(pts, pts_t)
    adj = adj8.view(jnp.bool_)
    return adj, pts, fts
